# baseline (speedup 1.0000x reference)
import jax
import jax.numpy as jnp
from jax.experimental import pallas as pl

_B = 512
_N = 32
_T = 21
_AD = 11
_S = 5
_SD = 10
_L = 70
_P = 20
_LD = 12
_R = 25
_H = 256
_ACT = 8
_PRED = 10
_TOK = _N + _S + _L + _ACT + _PRED
_BB = 8
_KA = _T * _AD
_KL = _P * _LD
_KLA = _KL + 3


def _body(nb_ref, st_ref, ln_ref, rt_ref,
          Wa_ref, ba_ref, temb_ref, Ws_ref, bs_ref, Wl_ref, bl_ref,
          tremb_ref, Wr_ref, br_ref, Wp_ref, bp_ref,
          enc_a_ref, ln_enc_ref, mask_ref, rcond_ref, tdist_ref):
    f32 = jnp.float32
    Wp = Wp_ref[...]
    bp = bp_ref[...]

    nbf = nb_ref[...]
    valid_nb = (jnp.sum(jnp.abs(nbf), axis=1, keepdims=True) > 0.0).astype(f32)
    tidx = (jnp.abs(nbf[:, 230:231]) * 997.0).astype(jnp.int32) % 5
    oh_t = (tidx == jax.lax.broadcasted_iota(jnp.int32, (_BB * _N, 5), 1)).astype(f32)
    enc_nb = (jnp.dot(nbf, Wa_ref[...], preferred_element_type=f32)
              + ba_ref[...]
              + jnp.dot(oh_t, temb_ref[...], preferred_element_type=f32))
    pe_nb = jnp.dot(nbf[:, 220:227], Wp, preferred_element_type=f32) + bp
    enc_nb = enc_nb + valid_nb * pe_nb

    stf = st_ref[...]
    valid_st = (jnp.sum(jnp.abs(stf), axis=1, keepdims=True) > 0.0).astype(f32)
    enc_st = jnp.dot(stf, Ws_ref[...], preferred_element_type=f32) + bs_ref[...]
    pe_st = jnp.dot(stf[:, 0:7], Wp, preferred_element_type=f32) + bp
    enc_st = enc_st + valid_st * pe_st

    lnf = ln_ref[...]
    valid_ln = (jnp.sum(jnp.abs(lnf[:, 0:_KL]), axis=1, keepdims=True) > 0.0).astype(f32)
    tr_idx = (jnp.abs(lnf[:, 11:12]) * 997.0).astype(jnp.int32) % 4
    oh_tr = (tr_idx == jax.lax.broadcasted_iota(jnp.int32, (_BB * _L, 4), 1)).astype(f32)
    enc_ln = (jnp.dot(lnf, Wl_ref[...], preferred_element_type=f32)
              + bl_ref[...]
              + jnp.dot(oh_tr, tremb_ref[...], preferred_element_type=f32))
    pe_ln = jnp.dot(lnf[:, 120:127], Wp, preferred_element_type=f32) + bp
    enc_ln = enc_ln + valid_ln * pe_ln

    h = jnp.tanh(jnp.dot(rt_ref[...], Wr_ref[...], preferred_element_type=f32)
                 + br_ref[...])
    row = jax.lax.broadcasted_iota(jnp.int32, (_BB, _BB * _R), 1)
    grp = jax.lax.broadcasted_iota(jnp.int32, (_BB, _BB * _R), 0)
    sel = (row // _R == grp).astype(f32) * (1.0 / _R)
    rcond_ref[...] = jnp.dot(sel, h, preferred_element_type=f32)

    nbx = nbf[:, 220:221]
    nby = nbf[:, 221:222]
    stx = stf[:, 0:1]
    sty = stf[:, 1:2]
    lnx = lnf[:, 120:121]
    lny = lnf[:, 121:122]
    ego_x = jnp.full((_ACT, 1), -0.5, f32)
    ego_y = jnp.zeros((_ACT, 1), f32)
    for k in range(_BB):
        enc_a_ref[k, 0:_N, :] = enc_nb[k * _N:(k + 1) * _N, :]
        enc_a_ref[k, _N:_N + _S, :] = enc_st[k * _S:(k + 1) * _S, :]
        ln_enc_ref[k, :, :] = enc_ln[k * _L:(k + 1) * _L, :]

        mcol = jnp.concatenate([
            valid_nb[k * _N:(k + 1) * _N, :],
            valid_st[k * _S:(k + 1) * _S, :],
            valid_ln[k * _L:(k + 1) * _L, :],
        ], axis=0)
        mask_ref[k:k + 1, 0:107] = jnp.transpose(mcol)

        xk = jnp.concatenate([
            nbx[k * _N:(k + 1) * _N, :],
            stx[k * _S:(k + 1) * _S, :],
            lnx[k * _L:(k + 1) * _L, :],
            ego_x,
            nbx[k * _N:k * _N + _PRED, :],
        ], axis=0)
        yk = jnp.concatenate([
            nby[k * _N:(k + 1) * _N, :],
            sty[k * _S:(k + 1) * _S, :],
            lny[k * _L:(k + 1) * _L, :],
            ego_y,
            nby[k * _N:k * _N + _PRED, :],
        ], axis=0)
        dx = xk - jnp.transpose(xk)
        dy = yk - jnp.transpose(yk)
        tdist_ref[k, :, :] = jnp.sqrt(dx * dx + dy * dy)


def kernel(neighbors, static, lanes, lanes_speed_limit, lanes_has_speed_limit,
           routes, W_agent, b_agent, type_emb, W_static, b_static, W_lane,
           b_lane, W_sl, b_sl, unknown_sl, traffic_emb, W_route, b_route,
           W_pos, b_pos):
    f32 = jnp.float32
    Bc = neighbors.shape[0]
    nb2 = neighbors.reshape(Bc * _N, _KA)
    st2 = static.reshape(Bc * _S, _SD)
    hsf = lanes_has_speed_limit.astype(f32)
    ln_aug = jnp.concatenate([
        lanes.reshape(Bc * _L, _KL),
        (lanes_speed_limit * hsf).reshape(Bc * _L, 1),
        hsf.reshape(Bc * _L, 1),
        (1.0 - hsf).reshape(Bc * _L, 1),
    ], axis=1)
    Wl_aug = jnp.concatenate(
        [W_lane, W_sl, b_sl[None, :], unknown_sl[None, :]], axis=0)
    rt2 = routes.reshape(Bc * _R, _KL)

    grid = Bc // _BB

    def bm(*shape):
        nd = len(shape)
        return pl.BlockSpec(shape, lambda i, nd=nd: (i,) + (0,) * (nd - 1))

    def full(*shape):
        nd = len(shape)
        return pl.BlockSpec(shape, lambda i, nd=nd: (0,) * nd)

    out = pl.pallas_call(
        _body,
        grid=(grid,),
        in_specs=[
            bm(_BB * _N, _KA), bm(_BB * _S, _SD), bm(_BB * _L, _KLA),
            bm(_BB * _R, _KL),
            full(_KA, _H), full(1, _H), full(5, _H), full(_SD, _H),
            full(1, _H), full(_KLA, _H), full(1, _H), full(4, _H),
            full(_KL, _H), full(1, _H), full(7, _H), full(1, _H),
        ],
        out_specs=[
            bm(_BB, _N + _S, _H), bm(_BB, _L, _H), bm(_BB, 128),
            bm(_BB, _H), bm(_BB, _TOK, _TOK),
        ],
        out_shape=[
            jax.ShapeDtypeStruct((Bc, _N + _S, _H), f32),
            jax.ShapeDtypeStruct((Bc, _L, _H), f32),
            jax.ShapeDtypeStruct((Bc, 128), f32),
            jax.ShapeDtypeStruct((Bc, _H), f32),
            jax.ShapeDtypeStruct((Bc, _TOK, _TOK), f32),
        ],
    )(nb2, st2, ln_aug, rt2,
      W_agent, b_agent.reshape(1, _H), type_emb, W_static,
      b_static.reshape(1, _H), Wl_aug, b_lane.reshape(1, _H), traffic_emb,
      W_route, b_route.reshape(1, _H), W_pos, b_pos.reshape(1, _H))

    enc_a, ln_enc, mask_pack, rcond, tdist = out
    mask_a = mask_pack[:, :_N + _S] > 0.5
    ln_valid = mask_pack[:, _N + _S:_N + _S + _L] > 0.5
    return (enc_a, ln_enc, mask_a, ln_valid, rcond, tdist)


# speedup vs baseline: 1.0448x; 1.0448x over previous
import jax
import jax.numpy as jnp
from jax.experimental import pallas as pl

_B = 512
_N = 32
_T = 21
_AD = 11
_S = 5
_SD = 10
_L = 70
_P = 20
_LD = 12
_R = 25
_H = 256
_ACT = 8
_PRED = 10
_TOK = _N + _S + _L + _ACT + _PRED
_BB = 8
_KA = _T * _AD
_KL = _P * _LD
_KLA = _KL + 3


def _body(nb_ref, st_ref, ln_ref, rt_ref, x_ref, y_ref,
          Wa_ref, ba_ref, temb_ref, Ws_ref, bs_ref, Wl_ref, bl_ref,
          tremb_ref, Wr_ref, br_ref, Wp_ref, bp_ref,
          enc_a_ref, ln_enc_ref, vnb_ref, vst_ref, vln_ref, rcond_ref,
          tdist_ref):
    f32 = jnp.float32
    Wp = Wp_ref[...]
    bp = bp_ref[...]

    nbf = nb_ref[...]
    valid_nb = (jnp.dot(jnp.abs(nbf), jnp.ones((_KA, 1), f32),
                        preferred_element_type=f32) > 0.0).astype(f32)
    tidx = (jnp.abs(nbf[:, 230:231]) * 997.0).astype(jnp.int32) % 5
    oh_t = (tidx == jax.lax.broadcasted_iota(jnp.int32, (_BB * _N, 5), 1)).astype(f32)
    enc_nb = (jnp.dot(nbf, Wa_ref[...], preferred_element_type=f32)
              + ba_ref[...]
              + jnp.dot(oh_t, temb_ref[...], preferred_element_type=f32))
    pe_nb = jnp.dot(nbf[:, 220:227], Wp, preferred_element_type=f32) + bp
    enc_nb = enc_nb + valid_nb * pe_nb
    vnb_ref[...] = valid_nb

    stf = st_ref[...]
    valid_st = (jnp.dot(jnp.abs(stf), jnp.ones((_SD, 1), f32),
                        preferred_element_type=f32) > 0.0).astype(f32)
    enc_st = jnp.dot(stf, Ws_ref[...], preferred_element_type=f32) + bs_ref[...]
    pe_st = jnp.dot(stf[:, 0:7], Wp, preferred_element_type=f32) + bp
    enc_st = enc_st + valid_st * pe_st
    vst_ref[...] = valid_st

    lnf = ln_ref[...]
    valid_ln = (jnp.dot(jnp.abs(lnf[:, 0:_KL]), jnp.ones((_KL, 1), f32),
                        preferred_element_type=f32) > 0.0).astype(f32)
    tr_idx = (jnp.abs(lnf[:, 11:12]) * 997.0).astype(jnp.int32) % 4
    oh_tr = (tr_idx == jax.lax.broadcasted_iota(jnp.int32, (_BB * _L, 4), 1)).astype(f32)
    enc_ln = (jnp.dot(lnf, Wl_ref[...], preferred_element_type=f32)
              + bl_ref[...]
              + jnp.dot(oh_tr, tremb_ref[...], preferred_element_type=f32))
    pe_ln = jnp.dot(lnf[:, 120:127], Wp, preferred_element_type=f32) + bp
    enc_ln = enc_ln + valid_ln * pe_ln
    vln_ref[...] = valid_ln

    h = jnp.tanh(jnp.dot(rt_ref[...], Wr_ref[...], preferred_element_type=f32)
                 + br_ref[...])
    row = jax.lax.broadcasted_iota(jnp.int32, (_BB, _BB * _R), 1)
    grp = jax.lax.broadcasted_iota(jnp.int32, (_BB, _BB * _R), 0)
    sel = (row // _R == grp).astype(f32) * (1.0 / _R)
    rcond_ref[...] = jnp.dot(sel, h, preferred_element_type=f32)

    x = x_ref[...]
    y = y_ref[...]
    xT = jnp.transpose(x)
    yT = jnp.transpose(y)
    for k in range(_BB):
        enc_a_ref[k, 0:_N, :] = enc_nb[k * _N:(k + 1) * _N, :]
        enc_a_ref[k, _N:_N + _S, :] = enc_st[k * _S:(k + 1) * _S, :]
        ln_enc_ref[k, :, :] = enc_ln[k * _L:(k + 1) * _L, :]

        dx = xT[:, k:k + 1] - x[k:k + 1, :]
        dy = yT[:, k:k + 1] - y[k:k + 1, :]
        d = jnp.sqrt(dx * dx + dy * dy)
        tdist_ref[k, :, :] = d[0:_TOK, 0:_TOK]


def kernel(neighbors, static, lanes, lanes_speed_limit, lanes_has_speed_limit,
           routes, W_agent, b_agent, type_emb, W_static, b_static, W_lane,
           b_lane, W_sl, b_sl, unknown_sl, traffic_emb, W_route, b_route,
           W_pos, b_pos):
    f32 = jnp.float32
    Bc = neighbors.shape[0]
    nb2 = neighbors.reshape(Bc * _N, _KA)
    st2 = static.reshape(Bc * _S, _SD)
    hsf = lanes_has_speed_limit.astype(f32)
    ln_aug = jnp.concatenate([
        lanes.reshape(Bc * _L, _KL),
        (lanes_speed_limit * hsf).reshape(Bc * _L, 1),
        hsf.reshape(Bc * _L, 1),
        (1.0 - hsf).reshape(Bc * _L, 1),
    ], axis=1)
    Wl_aug = jnp.concatenate(
        [W_lane, W_sl, b_sl[None, :], unknown_sl[None, :]], axis=0)
    rt2 = routes.reshape(Bc * _R, _KL)

    nb_last = neighbors[:, :, -1, 0:2]
    xloc = jnp.concatenate([
        nb_last[:, :, 0],
        static[:, :, 0],
        lanes[:, :, _P // 2, 0],
        jnp.full((Bc, _ACT), -0.5, f32),
        nb_last[:, :_PRED, 0],
    ], axis=1)
    yloc = jnp.concatenate([
        nb_last[:, :, 1],
        static[:, :, 1],
        lanes[:, :, _P // 2, 1],
        jnp.zeros((Bc, _ACT), f32),
        nb_last[:, :_PRED, 1],
    ], axis=1)
    xloc = jnp.pad(xloc, ((0, 0), (0, 128 - _TOK)))
    yloc = jnp.pad(yloc, ((0, 0), (0, 128 - _TOK)))

    grid = Bc // _BB

    def bm(*shape):
        nd = len(shape)
        return pl.BlockSpec(shape, lambda i, nd=nd: (i,) + (0,) * (nd - 1))

    def full(*shape):
        nd = len(shape)
        return pl.BlockSpec(shape, lambda i, nd=nd: (0,) * nd)

    out = pl.pallas_call(
        _body,
        grid=(grid,),
        in_specs=[
            bm(_BB * _N, _KA), bm(_BB * _S, _SD), bm(_BB * _L, _KLA),
            bm(_BB * _R, _KL), bm(_BB, 128), bm(_BB, 128),
            full(_KA, _H), full(1, _H), full(5, _H), full(_SD, _H),
            full(1, _H), full(_KLA, _H), full(1, _H), full(4, _H),
            full(_KL, _H), full(1, _H), full(7, _H), full(1, _H),
        ],
        out_specs=[
            bm(_BB, _N + _S, _H), bm(_BB, _L, _H),
            bm(_BB * _N, 1), bm(_BB * _S, 1), bm(_BB * _L, 1),
            bm(_BB, _H), bm(_BB, _TOK, _TOK),
        ],
        out_shape=[
            jax.ShapeDtypeStruct((Bc, _N + _S, _H), f32),
            jax.ShapeDtypeStruct((Bc, _L, _H), f32),
            jax.ShapeDtypeStruct((Bc * _N, 1), f32),
            jax.ShapeDtypeStruct((Bc * _S, 1), f32),
            jax.ShapeDtypeStruct((Bc * _L, 1), f32),
            jax.ShapeDtypeStruct((Bc, _H), f32),
            jax.ShapeDtypeStruct((Bc, _TOK, _TOK), f32),
        ],
    )(nb2, st2, ln_aug, rt2, xloc, yloc,
      W_agent, b_agent.reshape(1, _H), type_emb, W_static,
      b_static.reshape(1, _H), Wl_aug, b_lane.reshape(1, _H), traffic_emb,
      W_route, b_route.reshape(1, _H), W_pos, b_pos.reshape(1, _H))

    enc_a, ln_enc, vnb, vst, vln, rcond, tdist = out
    mask_a = jnp.concatenate(
        [vnb.reshape(Bc, _N), vst.reshape(Bc, _S)], axis=1) > 0.5
    ln_valid = vln.reshape(Bc, _L) > 0.5
    return (enc_a, ln_enc, mask_a, ln_valid, rcond, tdist)


# speedup vs baseline: 1.2590x; 1.2050x over previous
import jax
import jax.numpy as jnp
from jax.experimental import pallas as pl

_B = 512
_N = 32
_T = 21
_AD = 11
_S = 5
_SD = 10
_L = 70
_P = 20
_LD = 12
_R = 25
_H = 256
_ACT = 8
_PRED = 10
_TOK = _N + _S + _L + _ACT + _PRED
_BB = 16
_KA = _T * _AD
_KL = _P * _LD
_KLA = _KL + 3


def _body(nb_ref, st_ref, ln_ref, aux_ref, rt_ref, x_ref, y_ref,
          Wa_ref, ba_ref, temb_ref, Ws_ref, bs_ref, Wl_ref, bl_ref,
          Wsl3_ref, tremb_ref, Wr_ref, br_ref, Wp_ref, bp_ref,
          enc_a_ref, ln_enc_ref, vnb_ref, vst_ref, vln_ref, rcond_ref,
          tdist_ref):
    f32 = jnp.float32
    Wp = Wp_ref[...]
    bp = bp_ref[...]

    nbf = nb_ref[...]
    valid_nb = (jnp.dot(jnp.abs(nbf), jnp.ones((_KA, 1), f32),
                        preferred_element_type=f32) > 0.0).astype(f32)
    tidx = (jnp.abs(nbf[:, 230:231]) * 997.0).astype(jnp.int32) % 5
    oh_t = (tidx == jax.lax.broadcasted_iota(jnp.int32, (_BB * _N, 5), 1)).astype(f32)
    enc_nb = (jnp.dot(nbf, Wa_ref[...], preferred_element_type=f32)
              + ba_ref[...]
              + jnp.dot(oh_t, temb_ref[...], preferred_element_type=f32))
    pe_nb = jnp.dot(nbf[:, 220:227], Wp, preferred_element_type=f32) + bp
    enc_nb = enc_nb + valid_nb * pe_nb
    vnb_ref[...] = valid_nb

    stf = st_ref[...]
    valid_st = (jnp.dot(jnp.abs(stf), jnp.ones((_SD, 1), f32),
                        preferred_element_type=f32) > 0.0).astype(f32)
    enc_st = jnp.dot(stf, Ws_ref[...], preferred_element_type=f32) + bs_ref[...]
    pe_st = jnp.dot(stf[:, 0:7], Wp, preferred_element_type=f32) + bp
    enc_st = enc_st + valid_st * pe_st
    vst_ref[...] = valid_st

    lnf = ln_ref[...]
    valid_ln = (jnp.dot(jnp.abs(lnf), jnp.ones((_KL, 1), f32),
                        preferred_element_type=f32) > 0.0).astype(f32)
    tr_idx = (jnp.abs(lnf[:, 11:12]) * 997.0).astype(jnp.int32) % 4
    oh_tr = (tr_idx == jax.lax.broadcasted_iota(jnp.int32, (_BB * _L, 4), 1)).astype(f32)
    enc_ln = (jnp.dot(lnf, Wl_ref[...], preferred_element_type=f32)
              + bl_ref[...]
              + jnp.dot(aux_ref[...], Wsl3_ref[...], preferred_element_type=f32)
              + jnp.dot(oh_tr, tremb_ref[...], preferred_element_type=f32))
    pe_ln = jnp.dot(lnf[:, 120:127], Wp, preferred_element_type=f32) + bp
    enc_ln = enc_ln + valid_ln * pe_ln
    vln_ref[...] = valid_ln

    h = jnp.tanh(jnp.dot(rt_ref[...], Wr_ref[...], preferred_element_type=f32)
                 + br_ref[...])
    row = jax.lax.broadcasted_iota(jnp.int32, (_BB, _BB * _R), 1)
    grp = jax.lax.broadcasted_iota(jnp.int32, (_BB, _BB * _R), 0)
    sel = (row // _R == grp).astype(f32) * (1.0 / _R)
    rcond_ref[...] = jnp.dot(sel, h, preferred_element_type=f32)

    x = x_ref[...]
    y = y_ref[...]
    xT = jnp.transpose(x)
    yT = jnp.transpose(y)
    for k in range(_BB):
        enc_a_ref[k, 0:_N, :] = enc_nb[k * _N:(k + 1) * _N, :]
        enc_a_ref[k, _N:_N + _S, :] = enc_st[k * _S:(k + 1) * _S, :]
        ln_enc_ref[k, :, :] = enc_ln[k * _L:(k + 1) * _L, :]

        dx = xT[:, k:k + 1] - x[k:k + 1, :]
        dy = yT[:, k:k + 1] - y[k:k + 1, :]
        d = jnp.sqrt(dx * dx + dy * dy)
        tdist_ref[k, :, :] = d[0:_TOK, 0:_TOK]


def kernel(neighbors, static, lanes, lanes_speed_limit, lanes_has_speed_limit,
           routes, W_agent, b_agent, type_emb, W_static, b_static, W_lane,
           b_lane, W_sl, b_sl, unknown_sl, traffic_emb, W_route, b_route,
           W_pos, b_pos):
    f32 = jnp.float32
    Bc = neighbors.shape[0]
    nb2 = neighbors.reshape(Bc * _N, _KA)
    st2 = static.reshape(Bc * _S, _SD)
    hsf = lanes_has_speed_limit.astype(f32)
    ln2 = lanes.reshape(Bc * _L, _KL)
    aux = jnp.concatenate([
        lanes_speed_limit * hsf, hsf, 1.0 - hsf], axis=2).reshape(Bc * _L, 3)
    Wsl3 = jnp.concatenate(
        [W_sl, b_sl[None, :], unknown_sl[None, :]], axis=0)
    rt2 = routes.reshape(Bc * _R, _KL)

    nb_last = neighbors[:, :, -1, 0:2]
    xloc = jnp.concatenate([
        nb_last[:, :, 0],
        static[:, :, 0],
        lanes[:, :, _P // 2, 0],
        jnp.full((Bc, _ACT), -0.5, f32),
        nb_last[:, :_PRED, 0],
    ], axis=1)
    yloc = jnp.concatenate([
        nb_last[:, :, 1],
        static[:, :, 1],
        lanes[:, :, _P // 2, 1],
        jnp.zeros((Bc, _ACT), f32),
        nb_last[:, :_PRED, 1],
    ], axis=1)
    xloc = jnp.pad(xloc, ((0, 0), (0, 128 - _TOK)))
    yloc = jnp.pad(yloc, ((0, 0), (0, 128 - _TOK)))

    grid = Bc // _BB

    def bm(*shape):
        nd = len(shape)
        return pl.BlockSpec(shape, lambda i, nd=nd: (i,) + (0,) * (nd - 1))

    def full(*shape):
        nd = len(shape)
        return pl.BlockSpec(shape, lambda i, nd=nd: (0,) * nd)

    out = pl.pallas_call(
        _body,
        grid=(grid,),
        in_specs=[
            bm(_BB * _N, _KA), bm(_BB * _S, _SD), bm(_BB * _L, _KL),
            bm(_BB * _L, 3), bm(_BB * _R, _KL), bm(_BB, 128), bm(_BB, 128),
            full(_KA, _H), full(1, _H), full(5, _H), full(_SD, _H),
            full(1, _H), full(_KL, _H), full(1, _H), full(3, _H),
            full(4, _H), full(_KL, _H), full(1, _H), full(7, _H),
            full(1, _H),
        ],
        out_specs=[
            bm(_BB, _N + _S, _H), bm(_BB, _L, _H),
            bm(_BB * _N, 1), bm(_BB * _S, 1), bm(_BB * _L, 1),
            bm(_BB, _H), bm(_BB, _TOK, _TOK),
        ],
        out_shape=[
            jax.ShapeDtypeStruct((Bc, _N + _S, _H), f32),
            jax.ShapeDtypeStruct((Bc, _L, _H), f32),
            jax.ShapeDtypeStruct((Bc * _N, 1), f32),
            jax.ShapeDtypeStruct((Bc * _S, 1), f32),
            jax.ShapeDtypeStruct((Bc * _L, 1), f32),
            jax.ShapeDtypeStruct((Bc, _H), f32),
            jax.ShapeDtypeStruct((Bc, _TOK, _TOK), f32),
        ],
    )(nb2, st2, ln2, aux, rt2, xloc, yloc,
      W_agent, b_agent.reshape(1, _H), type_emb, W_static,
      b_static.reshape(1, _H), W_lane, b_lane.reshape(1, _H), Wsl3,
      traffic_emb, W_route, b_route.reshape(1, _H), W_pos,
      b_pos.reshape(1, _H))

    enc_a, ln_enc, vnb, vst, vln, rcond, tdist = out
    mask_a = jnp.concatenate(
        [vnb.reshape(Bc, _N), vst.reshape(Bc, _S)], axis=1) > 0.5
    ln_valid = vln.reshape(Bc, _L) > 0.5
    return (enc_a, ln_enc, mask_a, ln_valid, rcond, tdist)


# speedup vs baseline: 1.2814x; 1.0178x over previous
import jax
import jax.numpy as jnp
from jax.experimental import pallas as pl

_B = 512
_N = 32
_T = 21
_AD = 11
_S = 5
_SD = 10
_L = 70
_P = 20
_LD = 12
_R = 25
_H = 256
_ACT = 8
_PRED = 10
_TOK = _N + _S + _L + _ACT + _PRED
_BB = 32
_KA = _T * _AD
_KL = _P * _LD
_KLA = _KL + 3


def _body(nb_ref, st_ref, ln_ref, aux_ref, rt_ref, x_ref, y_ref,
          Wa_ref, ba_ref, temb_ref, Ws_ref, bs_ref, Wl_ref, bl_ref,
          Wsl3_ref, tremb_ref, Wr_ref, br_ref, Wp_ref, bp_ref,
          enc_a_ref, ln_enc_ref, vnb_ref, vst_ref, vln_ref, rcond_ref,
          tdist_ref):
    f32 = jnp.float32
    Wp = Wp_ref[...]
    bp = bp_ref[...]

    nbf = nb_ref[...]
    valid_nb = (jnp.dot(jnp.abs(nbf), jnp.ones((_KA, 1), f32),
                        preferred_element_type=f32) > 0.0).astype(f32)
    tidx = (jnp.abs(nbf[:, 230:231]) * 997.0).astype(jnp.int32) % 5
    oh_t = (tidx == jax.lax.broadcasted_iota(jnp.int32, (_BB * _N, 5), 1)).astype(f32)
    enc_nb = (jnp.dot(nbf, Wa_ref[...], preferred_element_type=f32)
              + ba_ref[...]
              + jnp.dot(oh_t, temb_ref[...], preferred_element_type=f32))
    pe_nb = jnp.dot(nbf[:, 220:227], Wp, preferred_element_type=f32) + bp
    enc_nb = enc_nb + valid_nb * pe_nb
    vnb_ref[...] = valid_nb

    stf = st_ref[...]
    valid_st = (jnp.dot(jnp.abs(stf), jnp.ones((_SD, 1), f32),
                        preferred_element_type=f32) > 0.0).astype(f32)
    enc_st = jnp.dot(stf, Ws_ref[...], preferred_element_type=f32) + bs_ref[...]
    pe_st = jnp.dot(stf[:, 0:7], Wp, preferred_element_type=f32) + bp
    enc_st = enc_st + valid_st * pe_st
    vst_ref[...] = valid_st

    lnf = ln_ref[...]
    valid_ln = (jnp.dot(jnp.abs(lnf), jnp.ones((_KL, 1), f32),
                        preferred_element_type=f32) > 0.0).astype(f32)
    tr_idx = (jnp.abs(lnf[:, 11:12]) * 997.0).astype(jnp.int32) % 4
    oh_tr = (tr_idx == jax.lax.broadcasted_iota(jnp.int32, (_BB * _L, 4), 1)).astype(f32)
    enc_ln = (jnp.dot(lnf, Wl_ref[...], preferred_element_type=f32)
              + bl_ref[...]
              + jnp.dot(aux_ref[...], Wsl3_ref[...], preferred_element_type=f32)
              + jnp.dot(oh_tr, tremb_ref[...], preferred_element_type=f32))
    pe_ln = jnp.dot(lnf[:, 120:127], Wp, preferred_element_type=f32) + bp
    enc_ln = enc_ln + valid_ln * pe_ln
    vln_ref[...] = valid_ln

    h = jnp.tanh(jnp.dot(rt_ref[...], Wr_ref[...], preferred_element_type=f32)
                 + br_ref[...])
    row = jax.lax.broadcasted_iota(jnp.int32, (_BB, _BB * _R), 1)
    grp = jax.lax.broadcasted_iota(jnp.int32, (_BB, _BB * _R), 0)
    sel = (row // _R == grp).astype(f32) * (1.0 / _R)
    rcond_ref[...] = jnp.dot(sel, h, preferred_element_type=f32)

    x = x_ref[...]
    y = y_ref[...]
    xT = jnp.transpose(x)
    yT = jnp.transpose(y)
    for k in range(_BB):
        enc_a_ref[k, 0:_N, :] = enc_nb[k * _N:(k + 1) * _N, :]
        enc_a_ref[k, _N:_N + _S, :] = enc_st[k * _S:(k + 1) * _S, :]
        ln_enc_ref[k, :, :] = enc_ln[k * _L:(k + 1) * _L, :]

        dx = xT[:, k:k + 1] - x[k:k + 1, :]
        dy = yT[:, k:k + 1] - y[k:k + 1, :]
        d = jnp.sqrt(dx * dx + dy * dy)
        tdist_ref[k, :, :] = d[0:_TOK, 0:_TOK]


def kernel(neighbors, static, lanes, lanes_speed_limit, lanes_has_speed_limit,
           routes, W_agent, b_agent, type_emb, W_static, b_static, W_lane,
           b_lane, W_sl, b_sl, unknown_sl, traffic_emb, W_route, b_route,
           W_pos, b_pos):
    f32 = jnp.float32
    Bc = neighbors.shape[0]
    nb2 = neighbors.reshape(Bc * _N, _KA)
    st2 = static.reshape(Bc * _S, _SD)
    hsf = lanes_has_speed_limit.astype(f32)
    ln2 = lanes.reshape(Bc * _L, _KL)
    aux = jnp.concatenate([
        lanes_speed_limit * hsf, hsf, 1.0 - hsf], axis=2).reshape(Bc * _L, 3)
    Wsl3 = jnp.concatenate(
        [W_sl, b_sl[None, :], unknown_sl[None, :]], axis=0)
    rt2 = routes.reshape(Bc * _R, _KL)

    nb_last = neighbors[:, :, -1, 0:2]
    xloc = jnp.concatenate([
        nb_last[:, :, 0],
        static[:, :, 0],
        lanes[:, :, _P // 2, 0],
        jnp.full((Bc, _ACT), -0.5, f32),
        nb_last[:, :_PRED, 0],
    ], axis=1)
    yloc = jnp.concatenate([
        nb_last[:, :, 1],
        static[:, :, 1],
        lanes[:, :, _P // 2, 1],
        jnp.zeros((Bc, _ACT), f32),
        nb_last[:, :_PRED, 1],
    ], axis=1)
    xloc = jnp.pad(xloc, ((0, 0), (0, 128 - _TOK)))
    yloc = jnp.pad(yloc, ((0, 0), (0, 128 - _TOK)))

    grid = Bc // _BB

    def bm(*shape):
        nd = len(shape)
        return pl.BlockSpec(shape, lambda i, nd=nd: (i,) + (0,) * (nd - 1))

    def full(*shape):
        nd = len(shape)
        return pl.BlockSpec(shape, lambda i, nd=nd: (0,) * nd)

    out = pl.pallas_call(
        _body,
        grid=(grid,),
        in_specs=[
            bm(_BB * _N, _KA), bm(_BB * _S, _SD), bm(_BB * _L, _KL),
            bm(_BB * _L, 3), bm(_BB * _R, _KL), bm(_BB, 128), bm(_BB, 128),
            full(_KA, _H), full(1, _H), full(5, _H), full(_SD, _H),
            full(1, _H), full(_KL, _H), full(1, _H), full(3, _H),
            full(4, _H), full(_KL, _H), full(1, _H), full(7, _H),
            full(1, _H),
        ],
        out_specs=[
            bm(_BB, _N + _S, _H), bm(_BB, _L, _H),
            bm(_BB * _N, 1), bm(_BB * _S, 1), bm(_BB * _L, 1),
            bm(_BB, _H), bm(_BB, _TOK, _TOK),
        ],
        out_shape=[
            jax.ShapeDtypeStruct((Bc, _N + _S, _H), f32),
            jax.ShapeDtypeStruct((Bc, _L, _H), f32),
            jax.ShapeDtypeStruct((Bc * _N, 1), f32),
            jax.ShapeDtypeStruct((Bc * _S, 1), f32),
            jax.ShapeDtypeStruct((Bc * _L, 1), f32),
            jax.ShapeDtypeStruct((Bc, _H), f32),
            jax.ShapeDtypeStruct((Bc, _TOK, _TOK), f32),
        ],
    )(nb2, st2, ln2, aux, rt2, xloc, yloc,
      W_agent, b_agent.reshape(1, _H), type_emb, W_static,
      b_static.reshape(1, _H), W_lane, b_lane.reshape(1, _H), Wsl3,
      traffic_emb, W_route, b_route.reshape(1, _H), W_pos,
      b_pos.reshape(1, _H))

    enc_a, ln_enc, vnb, vst, vln, rcond, tdist = out
    mask_a = jnp.concatenate(
        [vnb.reshape(Bc, _N), vst.reshape(Bc, _S)], axis=1) > 0.5
    ln_valid = vln.reshape(Bc, _L) > 0.5
    return (enc_a, ln_enc, mask_a, ln_valid, rcond, tdist)
